# baseline (speedup 1.0000x reference)
import functools

import jax
import jax.numpy as jnp
from jax import lax
from jax.experimental import pallas as pl
from jax.experimental.pallas import tpu as pltpu
from jax.experimental.pallas import tpu_sc as plsc

N = 10000
D = 128
E = 320000
G = 256

NC = 2
NS = 16
NW = NC * NS

NPAD = 10240
CNK = 32
NCH = 320
EPW = CNK * NCH
E_PAD = EPW * NW
SENT = 10200
NBUF = 8
SCH = 64
NSUP = NCH // SCH

RB = 1280
GRID = NPAD // RB
DEGW = 16

_mesh = plsc.VectorSubcoreMesh(
    core_axis_name="c", subcore_axis_name="s", num_cores=NC, num_subcores=NS)


@functools.partial(
    pl.kernel,
    out_type=jax.ShapeDtypeStruct((NC, NPAD), jnp.float32),
    mesh=_mesh,
    scratch_types=[
        pltpu.VMEM((NCH, CNK), jnp.int32),
        pltpu.VMEM((CNK,), jnp.float32),
        pltpu.VMEM((NPAD // NS,), jnp.float32),
        pltpu.VMEM_SHARED((NPAD,), jnp.float32),
        pltpu.SemaphoreType.DMA,
    ],
)
def _sc_deg(dst_hbm, out_hbm, idx_v, ones_v, z_v, acc_sh, sem):
    cid = lax.axis_index("c")
    sid = lax.axis_index("s")
    wid = cid * NS + sid

    @pl.loop(0, CNK, step=16)
    def _(i):
        ones_v[pl.ds(i, 16)] = jnp.ones((16,), jnp.float32)

    @pl.loop(0, NPAD // NS, step=16)
    def _(i):
        z_v[pl.ds(i, 16)] = jnp.zeros((16,), jnp.float32)

    pltpu.sync_copy(z_v, acc_sh.at[pl.ds(sid * (NPAD // NS), NPAD // NS)])
    pltpu.sync_copy(dst_hbm.at[wid], idx_v)
    plsc.subcore_barrier()

    @pl.loop(0, NCH)
    def _(j):
        pltpu.async_copy(ones_v, acc_sh.at[idx_v.at[j]], sem, add=True)

    @pl.loop(0, NCH)
    def _(j):
        pltpu.make_async_copy(ones_v, acc_sh.at[pl.ds(0, CNK)], sem).wait()
    plsc.subcore_barrier()

    s0 = sid * (NPAD // NS)
    pltpu.sync_copy(acc_sh.at[pl.ds(s0, NPAD // NS)],
                    out_hbm.at[cid, pl.ds(s0, NPAD // NS)])


@functools.partial(
    pl.kernel,
    out_type=jax.ShapeDtypeStruct((NC, NPAD, D), jnp.float32),
    mesh=_mesh,
    scratch_types=[
        pltpu.VMEM((SCH, CNK), jnp.int32),
        pltpu.VMEM((SCH, CNK), jnp.int32),
        pltpu.VMEM((NBUF, CNK, D), jnp.float32),
        pltpu.VMEM_SHARED((NPAD, D), jnp.float32),
        [pltpu.SemaphoreType.DMA] * NBUF,
        [pltpu.SemaphoreType.DMA] * NBUF,
    ],
)
def _sc_edge(g_hbm, src_hbm, dst_hbm, out_hbm, si_v, di_v, rows_v, acc_sh,
             gsems, ssems):
    cid = lax.axis_index("c")
    sid = lax.axis_index("s")
    wid = cid * NS + sid

    @pl.loop(0, CNK)
    def _(r):
        for c in range(D // 16):
            rows_v[0, r, pl.ds(c * 16, 16)] = jnp.zeros((16,), jnp.float32)

    @pl.loop(0, NPAD // NS, step=CNK)
    def _(r0):
        pltpu.sync_copy(rows_v.at[0],
                        acc_sh.at[pl.ds(sid * (NPAD // NS) + r0, CNK)])
    plsc.subcore_barrier()

    def start_gather(jj, b):
        pltpu.async_copy(g_hbm.at[si_v.at[jj]], rows_v.at[b], gsems[b])

    def start_scatter(jj, b):
        pltpu.async_copy(rows_v.at[b], acc_sh.at[di_v.at[jj]], ssems[b],
                         add=True)

    def wait(sem, b):
        pltpu.make_async_copy(g_hbm.at[pl.ds(0, CNK)], rows_v.at[b],
                              sem).wait()

    @pl.loop(0, NSUP)
    def _(s):
        pltpu.sync_copy(src_hbm.at[wid, pl.ds(s * SCH, SCH)], si_v)
        pltpu.sync_copy(dst_hbm.at[wid, pl.ds(s * SCH, SCH)], di_v)
        for b in range(NBUF):
            start_gather(b, b)

        @pl.loop(0, SCH - NBUF, step=NBUF)
        def _(jj):
            for b in range(NBUF):
                wait(gsems[b], b)
                start_scatter(jj + b, b)
            for b in range(NBUF):
                wait(ssems[b], b)
                start_gather(jj + NBUF + b, b)

        for b in range(NBUF):
            wait(gsems[b], b)
            start_scatter(SCH - NBUF + b, b)
        for b in range(NBUF):
            wait(ssems[b], b)

    plsc.subcore_barrier()

    s0 = sid * (NPAD // NS)
    pltpu.sync_copy(acc_sh.at[pl.ds(s0, NPAD // NS)],
                    out_hbm.at[cid, pl.ds(s0, NPAD // NS)])


def _ln_matmul(h, scale, bias, w, dinv):
    mu = jnp.mean(h, axis=-1, keepdims=True)
    hc = h - mu
    var = jnp.mean(hc * hc, axis=-1, keepdims=True)
    ln = hc * lax.rsqrt(var + 1e-5) * scale + bias
    return jnp.dot(ln, w, preferred_element_type=jnp.float32) * dinv


def _dinv_of(deg_ref):
    d = deg_ref[0] + deg_ref[1] + 1.0
    return lax.rsqrt(d)[:, None]


def _pre0_body(x_ref, deg_ref, s_ref, b_ref, w_ref, o_ref):
    o_ref[...] = _ln_matmul(x_ref[...], s_ref[...], b_ref[...], w_ref[...],
                            _dinv_of(deg_ref))


def _layer_body(sp_ref, g_ref, deg_ref, bp_ref, s_ref, b_ref, w_ref, o_ref):
    dinv = _dinv_of(deg_ref)
    h = jnp.maximum((sp_ref[0] + sp_ref[1] + g_ref[...]) * dinv + bp_ref[...],
                    0.0)
    o_ref[...] = _ln_matmul(h, s_ref[...], b_ref[...], w_ref[...], dinv)


def _final_body(sp_ref, g_ref, deg_ref, bp_ref, batch_ref, lw_ref, lb_ref,
                o_ref, pooled, cnt):
    i = pl.program_id(0)

    @pl.when(i == 0)
    def _():
        pooled[...] = jnp.zeros_like(pooled)
        cnt[...] = jnp.zeros_like(cnt)

    dinv = _dinv_of(deg_ref)
    h = jnp.maximum((sp_ref[0] + sp_ref[1] + g_ref[...]) * dinv + bp_ref[...],
                    0.0)
    gids = lax.broadcasted_iota(jnp.int32, (G, 128), 0)
    for r in range(RB // 128):
        mask = (gids == batch_ref[0, r, :][None, :]).astype(jnp.float32)
        pooled[...] += jnp.dot(mask, h[r * 128:(r + 1) * 128, :],
                               preferred_element_type=jnp.float32)
        cnt[...] += jnp.sum(mask, axis=1, keepdims=True)

    @pl.when(i == GRID - 1)
    def _():
        p = pooled[...] / jnp.maximum(cnt[...], 1.0)
        o_ref[...] = (jnp.dot(p, lw_ref[...], preferred_element_type=jnp.float32)
                      + lb_ref[...])


def _row_spec():
    return pl.BlockSpec((RB, D), lambda i: (i, 0))


def _part_spec():
    return pl.BlockSpec((NC, RB, D), lambda i: (0, i, 0))


def _deg_spec():
    return pl.BlockSpec((NC, RB), lambda i: (0, i))


def _vec_spec():
    return pl.BlockSpec((1, D), lambda i: (0, 0))


def _mat_spec():
    return pl.BlockSpec((D, D), lambda i: (0, 0))


_pre0_call = pl.pallas_call(
    _pre0_body,
    grid=(GRID,),
    in_specs=[_row_spec(), _deg_spec(), _vec_spec(), _vec_spec(), _mat_spec()],
    out_specs=_row_spec(),
    out_shape=jax.ShapeDtypeStruct((NPAD, D), jnp.float32),
)

_layer_call = pl.pallas_call(
    _layer_body,
    grid=(GRID,),
    in_specs=[_part_spec(), _row_spec(), _deg_spec(), _vec_spec(), _vec_spec(),
              _vec_spec(), _mat_spec()],
    out_specs=_row_spec(),
    out_shape=jax.ShapeDtypeStruct((NPAD, D), jnp.float32),
)

_final_call = pl.pallas_call(
    _final_body,
    grid=(GRID,),
    in_specs=[_part_spec(), _row_spec(), _deg_spec(), _vec_spec(),
              pl.BlockSpec((1, RB // 128, 128), lambda i: (i, 0, 0)),
              _mat_spec(), _vec_spec()],
    out_specs=pl.BlockSpec((G, D), lambda i: (0, 0)),
    out_shape=jax.ShapeDtypeStruct((G, D), jnp.float32),
    scratch_shapes=[pltpu.VMEM((G, D), jnp.float32),
                    pltpu.VMEM((G, 1), jnp.float32)],
)


def kernel(x, edge_index, batch, ln_scale_0, ln_bias_0, W_0, b_0, ln_scale_1,
           ln_bias_1, W_1, b_1, ln_scale_2, ln_bias_2, W_2, b_2, lin_W, lin_b):
    pad_i = jnp.arange(E_PAD - E, dtype=jnp.int32)
    src = jnp.concatenate([edge_index[0], (pad_i * 131) % N])
    dst = jnp.concatenate([edge_index[1], N + (pad_i % (NPAD - N - 8))])
    src3 = src.reshape(NCH, NW, CNK).transpose(1, 0, 2)
    dst3 = dst.reshape(NCH, NW, CNK).transpose(1, 0, 2)
    x_pad = jnp.pad(x, ((0, NPAD - N), (0, 0)))
    batch3d = jnp.pad(batch, (0, NPAD - N),
                      constant_values=G + 7).reshape(GRID, RB // 128, 128)

    deg = _sc_deg(dst3)

    params = [(ln_scale_0.reshape(1, D), ln_bias_0.reshape(1, D), W_0,
               b_0.reshape(1, D)),
              (ln_scale_1.reshape(1, D), ln_bias_1.reshape(1, D), W_1,
               b_1.reshape(1, D)),
              (ln_scale_2.reshape(1, D), ln_bias_2.reshape(1, D), W_2,
               b_2.reshape(1, D))]

    g = _pre0_call(x_pad, deg, params[0][0], params[0][1], params[0][2])
    for l in (1, 2):
        s_part = _sc_edge(g, src3, dst3)
        g = _layer_call(s_part, g, deg, params[l - 1][3], params[l][0],
                        params[l][1], params[l][2])
    s_part = _sc_edge(g, src3, dst3)
    return _final_call(s_part, g, deg, params[2][3], batch3d, lin_W,
                       lin_b.reshape(1, D))


# speedup vs baseline: 1.0153x; 1.0153x over previous
import functools

import jax
import jax.numpy as jnp
from jax import lax
from jax.experimental import pallas as pl
from jax.experimental.pallas import tpu as pltpu
from jax.experimental.pallas import tpu_sc as plsc

N = 10000
D = 128
E = 320000
G = 256

NC = 2
NS = 16
NW = NC * NS

NPAD = 10240
CNK = 64
NCH = 160
EPW = CNK * NCH
E_PAD = EPW * NW
SENT = 10200
NBUF = 4
SCH = 32
NSUP = NCH // SCH

RB = 1280
GRID = NPAD // RB
DEGW = 16

_mesh = plsc.VectorSubcoreMesh(
    core_axis_name="c", subcore_axis_name="s", num_cores=NC, num_subcores=NS)


@functools.partial(
    pl.kernel,
    out_type=jax.ShapeDtypeStruct((NC, NPAD), jnp.float32),
    mesh=_mesh,
    scratch_types=[
        pltpu.VMEM((NCH, CNK), jnp.int32),
        pltpu.VMEM((CNK,), jnp.float32),
        pltpu.VMEM((NPAD // NS,), jnp.float32),
        pltpu.VMEM_SHARED((NPAD,), jnp.float32),
        pltpu.SemaphoreType.DMA,
    ],
)
def _sc_deg(dst_hbm, out_hbm, idx_v, ones_v, z_v, acc_sh, sem):
    cid = lax.axis_index("c")
    sid = lax.axis_index("s")
    wid = cid * NS + sid

    @pl.loop(0, CNK, step=16)
    def _(i):
        ones_v[pl.ds(i, 16)] = jnp.ones((16,), jnp.float32)

    @pl.loop(0, NPAD // NS, step=16)
    def _(i):
        z_v[pl.ds(i, 16)] = jnp.zeros((16,), jnp.float32)

    pltpu.sync_copy(z_v, acc_sh.at[pl.ds(sid * (NPAD // NS), NPAD // NS)])
    pltpu.sync_copy(dst_hbm.at[wid], idx_v)
    plsc.subcore_barrier()

    @pl.loop(0, NCH)
    def _(j):
        pltpu.async_copy(ones_v, acc_sh.at[idx_v.at[j]], sem, add=True)

    @pl.loop(0, NCH)
    def _(j):
        pltpu.make_async_copy(ones_v, acc_sh.at[pl.ds(0, CNK)], sem).wait()
    plsc.subcore_barrier()

    s0 = sid * (NPAD // NS)
    pltpu.sync_copy(acc_sh.at[pl.ds(s0, NPAD // NS)],
                    out_hbm.at[cid, pl.ds(s0, NPAD // NS)])


@functools.partial(
    pl.kernel,
    out_type=jax.ShapeDtypeStruct((NC, NPAD, D), jnp.float32),
    mesh=_mesh,
    scratch_types=[
        pltpu.VMEM((SCH, CNK), jnp.int32),
        pltpu.VMEM((SCH, CNK), jnp.int32),
        pltpu.VMEM((NBUF, CNK, D), jnp.float32),
        pltpu.VMEM_SHARED((NPAD, D), jnp.float32),
        [pltpu.SemaphoreType.DMA] * NBUF,
        [pltpu.SemaphoreType.DMA] * NBUF,
    ],
)
def _sc_edge(g_hbm, src_hbm, dst_hbm, out_hbm, si_v, di_v, rows_v, acc_sh,
             gsems, ssems):
    cid = lax.axis_index("c")
    sid = lax.axis_index("s")
    wid = cid * NS + sid

    @pl.loop(0, CNK)
    def _(r):
        for c in range(D // 16):
            rows_v[0, r, pl.ds(c * 16, 16)] = jnp.zeros((16,), jnp.float32)

    @pl.loop(0, NPAD // NS, step=CNK)
    def _(r0):
        pltpu.sync_copy(rows_v.at[0],
                        acc_sh.at[pl.ds(sid * (NPAD // NS) + r0, CNK)])
    plsc.subcore_barrier()

    def start_gather(jj, b):
        pltpu.async_copy(g_hbm.at[si_v.at[jj]], rows_v.at[b], gsems[b])

    def start_scatter(jj, b):
        pltpu.async_copy(rows_v.at[b], acc_sh.at[di_v.at[jj]], ssems[b],
                         add=True)

    def wait(sem, b):
        pltpu.make_async_copy(g_hbm.at[pl.ds(0, CNK)], rows_v.at[b],
                              sem).wait()

    @pl.loop(0, NSUP)
    def _(s):
        pltpu.sync_copy(src_hbm.at[wid, pl.ds(s * SCH, SCH)], si_v)
        pltpu.sync_copy(dst_hbm.at[wid, pl.ds(s * SCH, SCH)], di_v)
        for b in range(NBUF):
            start_gather(b, b)

        @pl.loop(0, SCH - NBUF, step=NBUF)
        def _(jj):
            for b in range(NBUF):
                wait(gsems[b], b)
                start_scatter(jj + b, b)
            for b in range(NBUF):
                wait(ssems[b], b)
                start_gather(jj + NBUF + b, b)

        for b in range(NBUF):
            wait(gsems[b], b)
            start_scatter(SCH - NBUF + b, b)
        for b in range(NBUF):
            wait(ssems[b], b)

    plsc.subcore_barrier()

    s0 = sid * (NPAD // NS)
    pltpu.sync_copy(acc_sh.at[pl.ds(s0, NPAD // NS)],
                    out_hbm.at[cid, pl.ds(s0, NPAD // NS)])


def _ln_matmul(h, scale, bias, w, dinv):
    mu = jnp.mean(h, axis=-1, keepdims=True)
    hc = h - mu
    var = jnp.mean(hc * hc, axis=-1, keepdims=True)
    ln = hc * lax.rsqrt(var + 1e-5) * scale + bias
    return jnp.dot(ln, w, preferred_element_type=jnp.float32) * dinv


def _dinv_of(deg_ref):
    d = deg_ref[0] + deg_ref[1] + 1.0
    return lax.rsqrt(d)[:, None]


def _pre0_body(x_ref, deg_ref, s_ref, b_ref, w_ref, o_ref):
    o_ref[...] = _ln_matmul(x_ref[...], s_ref[...], b_ref[...], w_ref[...],
                            _dinv_of(deg_ref))


def _layer_body(sp_ref, g_ref, deg_ref, bp_ref, s_ref, b_ref, w_ref, o_ref):
    dinv = _dinv_of(deg_ref)
    h = jnp.maximum((sp_ref[0] + sp_ref[1] + g_ref[...]) * dinv + bp_ref[...],
                    0.0)
    o_ref[...] = _ln_matmul(h, s_ref[...], b_ref[...], w_ref[...], dinv)


def _final_body(sp_ref, g_ref, deg_ref, bp_ref, batch_ref, lw_ref, lb_ref,
                o_ref, pooled, cnt):
    i = pl.program_id(0)

    @pl.when(i == 0)
    def _():
        pooled[...] = jnp.zeros_like(pooled)
        cnt[...] = jnp.zeros_like(cnt)

    dinv = _dinv_of(deg_ref)
    h = jnp.maximum((sp_ref[0] + sp_ref[1] + g_ref[...]) * dinv + bp_ref[...],
                    0.0)
    gids = lax.broadcasted_iota(jnp.int32, (G, 128), 0)
    for r in range(RB // 128):
        mask = (gids == batch_ref[0, r, :][None, :]).astype(jnp.float32)
        pooled[...] += jnp.dot(mask, h[r * 128:(r + 1) * 128, :],
                               preferred_element_type=jnp.float32)
        cnt[...] += jnp.sum(mask, axis=1, keepdims=True)

    @pl.when(i == GRID - 1)
    def _():
        p = pooled[...] / jnp.maximum(cnt[...], 1.0)
        o_ref[...] = (jnp.dot(p, lw_ref[...], preferred_element_type=jnp.float32)
                      + lb_ref[...])


def _row_spec():
    return pl.BlockSpec((RB, D), lambda i: (i, 0))


def _part_spec():
    return pl.BlockSpec((NC, RB, D), lambda i: (0, i, 0))


def _deg_spec():
    return pl.BlockSpec((NC, RB), lambda i: (0, i))


def _vec_spec():
    return pl.BlockSpec((1, D), lambda i: (0, 0))


def _mat_spec():
    return pl.BlockSpec((D, D), lambda i: (0, 0))


_pre0_call = pl.pallas_call(
    _pre0_body,
    grid=(GRID,),
    in_specs=[_row_spec(), _deg_spec(), _vec_spec(), _vec_spec(), _mat_spec()],
    out_specs=_row_spec(),
    out_shape=jax.ShapeDtypeStruct((NPAD, D), jnp.float32),
)

_layer_call = pl.pallas_call(
    _layer_body,
    grid=(GRID,),
    in_specs=[_part_spec(), _row_spec(), _deg_spec(), _vec_spec(), _vec_spec(),
              _vec_spec(), _mat_spec()],
    out_specs=_row_spec(),
    out_shape=jax.ShapeDtypeStruct((NPAD, D), jnp.float32),
)

_final_call = pl.pallas_call(
    _final_body,
    grid=(GRID,),
    in_specs=[_part_spec(), _row_spec(), _deg_spec(), _vec_spec(),
              pl.BlockSpec((1, RB // 128, 128), lambda i: (i, 0, 0)),
              _mat_spec(), _vec_spec()],
    out_specs=pl.BlockSpec((G, D), lambda i: (0, 0)),
    out_shape=jax.ShapeDtypeStruct((G, D), jnp.float32),
    scratch_shapes=[pltpu.VMEM((G, D), jnp.float32),
                    pltpu.VMEM((G, 1), jnp.float32)],
)


def kernel(x, edge_index, batch, ln_scale_0, ln_bias_0, W_0, b_0, ln_scale_1,
           ln_bias_1, W_1, b_1, ln_scale_2, ln_bias_2, W_2, b_2, lin_W, lin_b):
    pad_i = jnp.arange(E_PAD - E, dtype=jnp.int32)
    src = jnp.concatenate([edge_index[0], (pad_i * 131) % N])
    dst = jnp.concatenate([edge_index[1], N + (pad_i % (NPAD - N - 8))])
    src3 = src.reshape(NCH, NW, CNK).transpose(1, 0, 2)
    dst3 = dst.reshape(NCH, NW, CNK).transpose(1, 0, 2)
    x_pad = jnp.pad(x, ((0, NPAD - N), (0, 0)))
    batch3d = jnp.pad(batch, (0, NPAD - N),
                      constant_values=G + 7).reshape(GRID, RB // 128, 128)

    deg = _sc_deg(dst3)

    params = [(ln_scale_0.reshape(1, D), ln_bias_0.reshape(1, D), W_0,
               b_0.reshape(1, D)),
              (ln_scale_1.reshape(1, D), ln_bias_1.reshape(1, D), W_1,
               b_1.reshape(1, D)),
              (ln_scale_2.reshape(1, D), ln_bias_2.reshape(1, D), W_2,
               b_2.reshape(1, D))]

    g = _pre0_call(x_pad, deg, params[0][0], params[0][1], params[0][2])
    for l in (1, 2):
        s_part = _sc_edge(g, src3, dst3)
        g = _layer_call(s_part, g, deg, params[l - 1][3], params[l][0],
                        params[l][1], params[l][2])
    s_part = _sc_edge(g, src3, dst3)
    return _final_call(s_part, g, deg, params[2][3], batch3d, lin_W,
                       lin_b.reshape(1, D))


# speedup vs baseline: 1.0465x; 1.0307x over previous
import functools

import jax
import jax.numpy as jnp
from jax import lax
from jax.experimental import pallas as pl
from jax.experimental.pallas import tpu as pltpu
from jax.experimental.pallas import tpu_sc as plsc

N = 10000
D = 128
E = 320000
G = 256

NC = 2
NS = 16
NW = NC * NS

NPAD = 10240
CNK = 64
NCH = 160
EPW = CNK * NCH
E_PAD = EPW * NW
SENT = 10200
NBUF = 4
SCH = 16
NSUP = NCH // SCH
NIB = 3

RB = 2048
GRID = NPAD // RB
DEGW = 16

_mesh = plsc.VectorSubcoreMesh(
    core_axis_name="c", subcore_axis_name="s", num_cores=NC, num_subcores=NS)


@functools.partial(
    pl.kernel,
    out_type=jax.ShapeDtypeStruct((NC, NPAD), jnp.float32),
    mesh=_mesh,
    scratch_types=[
        pltpu.VMEM((NCH, CNK), jnp.int32),
        pltpu.VMEM((CNK,), jnp.float32),
        pltpu.VMEM((NPAD // NS,), jnp.float32),
        pltpu.VMEM_SHARED((NPAD,), jnp.float32),
        pltpu.SemaphoreType.DMA,
    ],
)
def _sc_deg(dst_hbm, out_hbm, idx_v, ones_v, z_v, acc_sh, sem):
    cid = lax.axis_index("c")
    sid = lax.axis_index("s")
    wid = cid * NS + sid

    @pl.loop(0, CNK, step=16)
    def _(i):
        ones_v[pl.ds(i, 16)] = jnp.ones((16,), jnp.float32)

    @pl.loop(0, NPAD // NS, step=16)
    def _(i):
        z_v[pl.ds(i, 16)] = jnp.zeros((16,), jnp.float32)

    pltpu.sync_copy(z_v, acc_sh.at[pl.ds(sid * (NPAD // NS), NPAD // NS)])
    pltpu.sync_copy(dst_hbm.at[wid], idx_v)
    plsc.subcore_barrier()

    @pl.loop(0, NCH)
    def _(j):
        pltpu.async_copy(ones_v, acc_sh.at[idx_v.at[j]], sem, add=True)

    @pl.loop(0, NCH)
    def _(j):
        pltpu.make_async_copy(ones_v, acc_sh.at[pl.ds(0, CNK)], sem).wait()
    plsc.subcore_barrier()

    s0 = sid * (NPAD // NS)
    pltpu.sync_copy(acc_sh.at[pl.ds(s0, NPAD // NS)],
                    out_hbm.at[cid, pl.ds(s0, NPAD // NS)])


@functools.partial(
    pl.kernel,
    out_type=jax.ShapeDtypeStruct((NC, NPAD, D), jnp.float32),
    mesh=_mesh,
    scratch_types=[
        pltpu.VMEM((NIB, SCH, CNK), jnp.int32),
        pltpu.VMEM((NIB, SCH, CNK), jnp.int32),
        pltpu.VMEM((NBUF, CNK, D), jnp.float32),
        pltpu.VMEM_SHARED((NPAD, D), jnp.float32),
        [pltpu.SemaphoreType.DMA] * NBUF,
        [pltpu.SemaphoreType.DMA] * NBUF,
    ],
)
def _sc_edge(g_hbm, src_hbm, dst_hbm, out_hbm, si_v, di_v, rows_v, acc_sh,
             gsems, ssems):
    cid = lax.axis_index("c")
    sid = lax.axis_index("s")
    wid = cid * NS + sid

    def start_gather(c, b):
        pltpu.async_copy(g_hbm.at[si_v.at[(c // SCH) % NIB, c % SCH]],
                         rows_v.at[b], gsems[b])

    def start_scatter(c, b):
        pltpu.async_copy(rows_v.at[b],
                         acc_sh.at[di_v.at[(c // SCH) % NIB, c % SCH]],
                         ssems[b], add=True)

    def wait(sem, b):
        pltpu.make_async_copy(g_hbm.at[pl.ds(0, CNK)], rows_v.at[b],
                              sem).wait()

    def load_idx(s, bi):
        pltpu.sync_copy(src_hbm.at[wid, pl.ds(s * SCH, SCH)], si_v.at[bi])
        pltpu.sync_copy(dst_hbm.at[wid, pl.ds(s * SCH, SCH)], di_v.at[bi])

    load_idx(0, 0)
    for b in range(1, NBUF):
        start_gather(b, b)

    @pl.loop(0, CNK)
    def _(r):
        for c in range(D // 16):
            rows_v[0, r, pl.ds(c * 16, 16)] = jnp.zeros((16,), jnp.float32)

    @pl.loop(0, NPAD // NS, step=CNK)
    def _(r0):
        pltpu.sync_copy(rows_v.at[0],
                        acc_sh.at[pl.ds(sid * (NPAD // NS) + r0, CNK)])
    plsc.subcore_barrier()
    start_gather(0, 0)

    @pl.loop(0, NCH - NBUF, step=NBUF)
    def _(jj):
        @pl.when(jj % SCH == 0)
        def _():
            s1 = jj // SCH + 1

            @pl.when(s1 < NSUP)
            def _():
                load_idx(s1, s1 % NIB)

        for b in range(NBUF):
            wait(gsems[b], b)
            start_scatter(jj + b, b)
        for b in range(NBUF):
            wait(ssems[b], b)
            start_gather(jj + NBUF + b, b)

    for b in range(NBUF):
        wait(gsems[b], b)
        start_scatter(NCH - NBUF + b, b)
    for b in range(NBUF):
        wait(ssems[b], b)
    plsc.subcore_barrier()

    s0 = sid * (NPAD // NS)
    pltpu.sync_copy(acc_sh.at[pl.ds(s0, NPAD // NS)],
                    out_hbm.at[cid, pl.ds(s0, NPAD // NS)])


def _ln_matmul(h, scale, bias, w, dinv):
    mu = jnp.mean(h, axis=-1, keepdims=True)
    hc = h - mu
    var = jnp.mean(hc * hc, axis=-1, keepdims=True)
    ln = hc * lax.rsqrt(var + 1e-5) * scale + bias
    return jnp.dot(ln, w, preferred_element_type=jnp.float32) * dinv


def _dinv_of(deg_ref):
    d = deg_ref[0] + deg_ref[1] + 1.0
    return lax.rsqrt(d)[:, None]


def _pre0_body(x_ref, deg_ref, s_ref, b_ref, w_ref, o_ref):
    o_ref[...] = _ln_matmul(x_ref[...], s_ref[...], b_ref[...], w_ref[...],
                            _dinv_of(deg_ref))


def _layer_body(sp_ref, g_ref, deg_ref, bp_ref, s_ref, b_ref, w_ref, o_ref):
    dinv = _dinv_of(deg_ref)
    h = jnp.maximum((sp_ref[0] + sp_ref[1] + g_ref[...]) * dinv + bp_ref[...],
                    0.0)
    o_ref[...] = _ln_matmul(h, s_ref[...], b_ref[...], w_ref[...], dinv)


def _final_body(sp_ref, g_ref, deg_ref, bp_ref, batch_ref, lw_ref, lb_ref,
                o_ref, pooled, cnt):
    i = pl.program_id(0)

    @pl.when(i == 0)
    def _():
        pooled[...] = jnp.zeros_like(pooled)
        cnt[...] = jnp.zeros_like(cnt)

    dinv = _dinv_of(deg_ref)
    h = jnp.maximum((sp_ref[0] + sp_ref[1] + g_ref[...]) * dinv + bp_ref[...],
                    0.0)
    gids = lax.broadcasted_iota(jnp.int32, (G, 128), 0)
    for r in range(RB // 128):
        mask = (gids == batch_ref[0, r, :][None, :]).astype(jnp.float32)
        pooled[...] += jnp.dot(mask, h[r * 128:(r + 1) * 128, :],
                               preferred_element_type=jnp.float32)
        cnt[...] += jnp.sum(mask, axis=1, keepdims=True)

    @pl.when(i == GRID - 1)
    def _():
        p = pooled[...] / jnp.maximum(cnt[...], 1.0)
        o_ref[...] = (jnp.dot(p, lw_ref[...], preferred_element_type=jnp.float32)
                      + lb_ref[...])


def _row_spec():
    return pl.BlockSpec((RB, D), lambda i: (i, 0))


def _part_spec():
    return pl.BlockSpec((NC, RB, D), lambda i: (0, i, 0))


def _deg_spec():
    return pl.BlockSpec((NC, RB), lambda i: (0, i))


def _vec_spec():
    return pl.BlockSpec((1, D), lambda i: (0, 0))


def _mat_spec():
    return pl.BlockSpec((D, D), lambda i: (0, 0))


_pre0_call = pl.pallas_call(
    _pre0_body,
    grid=(GRID,),
    in_specs=[_row_spec(), _deg_spec(), _vec_spec(), _vec_spec(), _mat_spec()],
    out_specs=_row_spec(),
    out_shape=jax.ShapeDtypeStruct((NPAD, D), jnp.float32),
)

_layer_call = pl.pallas_call(
    _layer_body,
    grid=(GRID,),
    in_specs=[_part_spec(), _row_spec(), _deg_spec(), _vec_spec(), _vec_spec(),
              _vec_spec(), _mat_spec()],
    out_specs=_row_spec(),
    out_shape=jax.ShapeDtypeStruct((NPAD, D), jnp.float32),
)

_final_call = pl.pallas_call(
    _final_body,
    grid=(GRID,),
    in_specs=[_part_spec(), _row_spec(), _deg_spec(), _vec_spec(),
              pl.BlockSpec((1, RB // 128, 128), lambda i: (i, 0, 0)),
              _mat_spec(), _vec_spec()],
    out_specs=pl.BlockSpec((G, D), lambda i: (0, 0)),
    out_shape=jax.ShapeDtypeStruct((G, D), jnp.float32),
    scratch_shapes=[pltpu.VMEM((G, D), jnp.float32),
                    pltpu.VMEM((G, 1), jnp.float32)],
)


def kernel(x, edge_index, batch, ln_scale_0, ln_bias_0, W_0, b_0, ln_scale_1,
           ln_bias_1, W_1, b_1, ln_scale_2, ln_bias_2, W_2, b_2, lin_W, lin_b):
    pad_i = jnp.arange(E_PAD - E, dtype=jnp.int32)
    src = jnp.concatenate([edge_index[0], (pad_i * 131) % N])
    dst = jnp.concatenate([edge_index[1], N + (pad_i % (NPAD - N - 8))])
    src3 = src.reshape(NCH, NW, CNK).transpose(1, 0, 2)
    dst3 = dst.reshape(NCH, NW, CNK).transpose(1, 0, 2)
    x_pad = jnp.pad(x, ((0, NPAD - N), (0, 0)))
    batch3d = jnp.pad(batch, (0, NPAD - N),
                      constant_values=G + 7).reshape(GRID, RB // 128, 128)

    deg = _sc_deg(dst3)

    params = [(ln_scale_0.reshape(1, D), ln_bias_0.reshape(1, D), W_0,
               b_0.reshape(1, D)),
              (ln_scale_1.reshape(1, D), ln_bias_1.reshape(1, D), W_1,
               b_1.reshape(1, D)),
              (ln_scale_2.reshape(1, D), ln_bias_2.reshape(1, D), W_2,
               b_2.reshape(1, D))]

    g = _pre0_call(x_pad, deg, params[0][0], params[0][1], params[0][2])
    for l in (1, 2):
        s_part = _sc_edge(g, src3, dst3)
        g = _layer_call(s_part, g, deg, params[l - 1][3], params[l][0],
                        params[l][1], params[l][2])
    s_part = _sc_edge(g, src3, dst3)
    return _final_call(s_part, g, deg, params[2][3], batch3d, lin_W,
                       lin_b.reshape(1, D))


# speedup vs baseline: 1.0501x; 1.0034x over previous
import functools

import jax
import jax.numpy as jnp
from jax import lax
from jax.experimental import pallas as pl
from jax.experimental.pallas import tpu as pltpu
from jax.experimental.pallas import tpu_sc as plsc

N = 10000
D = 128
E = 320000
G = 256

NC = 2
NS = 16
NW = NC * NS

NPAD = 10240
CNK = 64
NCH = 160
EPW = CNK * NCH
E_PAD = EPW * NW
SENT = 10200
NBUF = 4
SCH = 16
NSUP = NCH // SCH
NIB = 3

RB = 2048
GRID = NPAD // RB
DEGW = 16

_mesh = plsc.VectorSubcoreMesh(
    core_axis_name="c", subcore_axis_name="s", num_cores=NC, num_subcores=NS)


@functools.partial(
    pl.kernel,
    out_type=jax.ShapeDtypeStruct((NC, NPAD), jnp.float32),
    mesh=_mesh,
    scratch_types=[
        pltpu.VMEM((NCH, CNK), jnp.int32),
        pltpu.VMEM((CNK,), jnp.float32),
        pltpu.VMEM((NPAD // NS,), jnp.float32),
        pltpu.VMEM_SHARED((NPAD,), jnp.float32),
        pltpu.SemaphoreType.DMA,
    ],
)
def _sc_deg(dst_hbm, out_hbm, idx_v, ones_v, z_v, acc_sh, sem):
    cid = lax.axis_index("c")
    sid = lax.axis_index("s")
    wid = cid * NS + sid

    @pl.loop(0, CNK, step=16)
    def _(i):
        ones_v[pl.ds(i, 16)] = jnp.ones((16,), jnp.float32)

    @pl.loop(0, NPAD // NS, step=16)
    def _(i):
        z_v[pl.ds(i, 16)] = jnp.zeros((16,), jnp.float32)

    pltpu.sync_copy(z_v, acc_sh.at[pl.ds(sid * (NPAD // NS), NPAD // NS)])
    pltpu.sync_copy(dst_hbm.at[wid], idx_v)
    plsc.subcore_barrier()

    @pl.loop(0, NCH)
    def _(j):
        pltpu.async_copy(ones_v, acc_sh.at[idx_v.at[j]], sem, add=True)

    @pl.loop(0, NCH)
    def _(j):
        pltpu.make_async_copy(ones_v, acc_sh.at[pl.ds(0, CNK)], sem).wait()
    plsc.subcore_barrier()

    s0 = sid * (NPAD // NS)
    pltpu.sync_copy(acc_sh.at[pl.ds(s0, NPAD // NS)],
                    out_hbm.at[cid, pl.ds(s0, NPAD // NS)])


@functools.partial(
    pl.kernel,
    out_type=jax.ShapeDtypeStruct((NC, NPAD, D), jnp.float32),
    mesh=_mesh,
    scratch_types=[
        pltpu.VMEM((NIB, SCH, CNK), jnp.int32),
        pltpu.VMEM((NIB, SCH, CNK), jnp.int32),
        pltpu.VMEM((NBUF, CNK, D), jnp.float32),
        pltpu.VMEM_SHARED((NPAD, D), jnp.float32),
        [pltpu.SemaphoreType.DMA] * NBUF,
        [pltpu.SemaphoreType.DMA] * NBUF,
    ],
)
def _sc_edge(g_hbm, src_hbm, dst_hbm, out_hbm, si_v, di_v, rows_v, acc_sh,
             gsems, ssems):
    cid = lax.axis_index("c")
    sid = lax.axis_index("s")
    wid = cid * NS + sid

    def start_gather(c, b):
        pltpu.async_copy(g_hbm.at[si_v.at[(c // SCH) % NIB, c % SCH]],
                         rows_v.at[b], gsems[b])

    def start_scatter(c, b):
        pltpu.async_copy(rows_v.at[b],
                         acc_sh.at[di_v.at[(c // SCH) % NIB, c % SCH]],
                         ssems[b], add=True)

    def wait(sem, b):
        pltpu.make_async_copy(g_hbm.at[pl.ds(0, CNK)], rows_v.at[b],
                              sem).wait()

    def load_idx(s, bi):
        pltpu.sync_copy(src_hbm.at[wid, pl.ds(s * SCH, SCH)], si_v.at[bi])
        pltpu.sync_copy(dst_hbm.at[wid, pl.ds(s * SCH, SCH)], di_v.at[bi])

    load_idx(0, 0)
    for b in range(1, NBUF):
        start_gather(b, b)

    @pl.loop(0, CNK)
    def _(r):
        for c in range(D // 16):
            rows_v[0, r, pl.ds(c * 16, 16)] = jnp.zeros((16,), jnp.float32)

    for k in range(NPAD // NS // CNK):
        pltpu.async_copy(rows_v.at[0],
                         acc_sh.at[pl.ds(sid * (NPAD // NS) + k * CNK, CNK)],
                         ssems[0])
    for k in range(NPAD // NS // CNK):
        pltpu.make_async_copy(g_hbm.at[pl.ds(0, CNK)], rows_v.at[0],
                              ssems[0]).wait()
    plsc.subcore_barrier()
    start_gather(0, 0)

    @pl.loop(0, NCH - NBUF, step=NBUF)
    def _(jj):
        @pl.when(jj % SCH == 0)
        def _():
            s1 = jj // SCH + 1

            @pl.when(s1 < NSUP)
            def _():
                load_idx(s1, s1 % NIB)

        for b in range(NBUF):
            wait(gsems[b], b)
            start_scatter(jj + b, b)
        for b in range(NBUF):
            wait(ssems[b], b)
            start_gather(jj + NBUF + b, b)

    for b in range(NBUF):
        wait(gsems[b], b)
        start_scatter(NCH - NBUF + b, b)
    for b in range(NBUF):
        wait(ssems[b], b)
    plsc.subcore_barrier()

    s0 = sid * (NPAD // NS)
    pltpu.sync_copy(acc_sh.at[pl.ds(s0, NPAD // NS)],
                    out_hbm.at[cid, pl.ds(s0, NPAD // NS)])


def _ln_matmul(h, scale, bias, w, dinv):
    mu = jnp.mean(h, axis=-1, keepdims=True)
    hc = h - mu
    var = jnp.mean(hc * hc, axis=-1, keepdims=True)
    ln = hc * lax.rsqrt(var + 1e-5) * scale + bias
    return jnp.dot(ln, w, preferred_element_type=jnp.float32) * dinv


def _dinv_of(deg_ref):
    d = deg_ref[0] + deg_ref[1] + 1.0
    return lax.rsqrt(d)[:, None]


def _pre0_body(x_ref, deg_ref, s_ref, b_ref, w_ref, o_ref):
    o_ref[...] = _ln_matmul(x_ref[...], s_ref[...], b_ref[...], w_ref[...],
                            _dinv_of(deg_ref))


def _layer_body(sp_ref, g_ref, deg_ref, bp_ref, s_ref, b_ref, w_ref, o_ref):
    dinv = _dinv_of(deg_ref)
    h = jnp.maximum((sp_ref[0] + sp_ref[1] + g_ref[...]) * dinv + bp_ref[...],
                    0.0)
    o_ref[...] = _ln_matmul(h, s_ref[...], b_ref[...], w_ref[...], dinv)


def _final_body(sp_ref, g_ref, deg_ref, bp_ref, batch_ref, lw_ref, lb_ref,
                o_ref, pooled, cnt):
    i = pl.program_id(0)

    @pl.when(i == 0)
    def _():
        pooled[...] = jnp.zeros_like(pooled)
        cnt[...] = jnp.zeros_like(cnt)

    dinv = _dinv_of(deg_ref)
    h = jnp.maximum((sp_ref[0] + sp_ref[1] + g_ref[...]) * dinv + bp_ref[...],
                    0.0)
    gids = lax.broadcasted_iota(jnp.int32, (G, 128), 0)
    for r in range(RB // 128):
        mask = (gids == batch_ref[0, r, :][None, :]).astype(jnp.float32)
        pooled[...] += jnp.dot(mask, h[r * 128:(r + 1) * 128, :],
                               preferred_element_type=jnp.float32)
        cnt[...] += jnp.sum(mask, axis=1, keepdims=True)

    @pl.when(i == GRID - 1)
    def _():
        p = pooled[...] / jnp.maximum(cnt[...], 1.0)
        o_ref[...] = (jnp.dot(p, lw_ref[...], preferred_element_type=jnp.float32)
                      + lb_ref[...])


def _row_spec():
    return pl.BlockSpec((RB, D), lambda i: (i, 0))


def _part_spec():
    return pl.BlockSpec((NC, RB, D), lambda i: (0, i, 0))


def _deg_spec():
    return pl.BlockSpec((NC, RB), lambda i: (0, i))


def _vec_spec():
    return pl.BlockSpec((1, D), lambda i: (0, 0))


def _mat_spec():
    return pl.BlockSpec((D, D), lambda i: (0, 0))


_pre0_call = pl.pallas_call(
    _pre0_body,
    grid=(GRID,),
    in_specs=[_row_spec(), _deg_spec(), _vec_spec(), _vec_spec(), _mat_spec()],
    out_specs=_row_spec(),
    out_shape=jax.ShapeDtypeStruct((NPAD, D), jnp.float32),
)

_layer_call = pl.pallas_call(
    _layer_body,
    grid=(GRID,),
    in_specs=[_part_spec(), _row_spec(), _deg_spec(), _vec_spec(), _vec_spec(),
              _vec_spec(), _mat_spec()],
    out_specs=_row_spec(),
    out_shape=jax.ShapeDtypeStruct((NPAD, D), jnp.float32),
)

_final_call = pl.pallas_call(
    _final_body,
    grid=(GRID,),
    in_specs=[_part_spec(), _row_spec(), _deg_spec(), _vec_spec(),
              pl.BlockSpec((1, RB // 128, 128), lambda i: (i, 0, 0)),
              _mat_spec(), _vec_spec()],
    out_specs=pl.BlockSpec((G, D), lambda i: (0, 0)),
    out_shape=jax.ShapeDtypeStruct((G, D), jnp.float32),
    scratch_shapes=[pltpu.VMEM((G, D), jnp.float32),
                    pltpu.VMEM((G, 1), jnp.float32)],
)


def kernel(x, edge_index, batch, ln_scale_0, ln_bias_0, W_0, b_0, ln_scale_1,
           ln_bias_1, W_1, b_1, ln_scale_2, ln_bias_2, W_2, b_2, lin_W, lin_b):
    pad_i = jnp.arange(E_PAD - E, dtype=jnp.int32)
    src = jnp.concatenate([edge_index[0], (pad_i * 131) % N])
    dst = jnp.concatenate([edge_index[1], N + (pad_i % (NPAD - N - 8))])
    src3 = src.reshape(NCH, NW, CNK).transpose(1, 0, 2)
    dst3 = dst.reshape(NCH, NW, CNK).transpose(1, 0, 2)
    x_pad = jnp.pad(x, ((0, NPAD - N), (0, 0)))
    batch3d = jnp.pad(batch, (0, NPAD - N),
                      constant_values=G + 7).reshape(GRID, RB // 128, 128)

    deg = _sc_deg(dst3)

    params = [(ln_scale_0.reshape(1, D), ln_bias_0.reshape(1, D), W_0,
               b_0.reshape(1, D)),
              (ln_scale_1.reshape(1, D), ln_bias_1.reshape(1, D), W_1,
               b_1.reshape(1, D)),
              (ln_scale_2.reshape(1, D), ln_bias_2.reshape(1, D), W_2,
               b_2.reshape(1, D))]

    g = _pre0_call(x_pad, deg, params[0][0], params[0][1], params[0][2])
    for l in (1, 2):
        s_part = _sc_edge(g, src3, dst3)
        g = _layer_call(s_part, g, deg, params[l - 1][3], params[l][0],
                        params[l][1], params[l][2])
    s_part = _sc_edge(g, src3, dst3)
    return _final_call(s_part, g, deg, params[2][3], batch3d, lin_W,
                       lin_b.reshape(1, D))


# speedup vs baseline: 1.0516x; 1.0014x over previous
import functools

import jax
import jax.numpy as jnp
from jax import lax
from jax.experimental import pallas as pl
from jax.experimental.pallas import tpu as pltpu
from jax.experimental.pallas import tpu_sc as plsc

N = 10000
D = 128
E = 320000
G = 256

NC = 2
NS = 16
NW = NC * NS

NPAD = 10240
CNK = 64
NCH = 160
EPW = CNK * NCH
E_PAD = EPW * NW
NBUF = 4
SCH = 16
NSUP = NCH // SCH
NIB = 3

RB = 2048
GRID = NPAD // RB

_mesh = plsc.VectorSubcoreMesh(
    core_axis_name="c", subcore_axis_name="s", num_cores=NC, num_subcores=NS)


@functools.partial(
    pl.kernel,
    out_type=jax.ShapeDtypeStruct((NC, NPAD), jnp.float32),
    mesh=_mesh,
    scratch_types=[
        pltpu.VMEM((NCH, CNK), jnp.int32),
        pltpu.VMEM((CNK,), jnp.float32),
        pltpu.VMEM((NPAD // NS,), jnp.float32),
        pltpu.VMEM_SHARED((NPAD,), jnp.float32),
        pltpu.SemaphoreType.DMA,
    ],
)
def _sc_deg(dst_hbm, out_hbm, idx_v, ones_v, z_v, acc_sh, sem):
    cid = lax.axis_index("c")
    sid = lax.axis_index("s")
    wid = cid * NS + sid

    @pl.loop(0, CNK, step=16)
    def _(i):
        ones_v[pl.ds(i, 16)] = jnp.ones((16,), jnp.float32)

    @pl.loop(0, NPAD // NS, step=16)
    def _(i):
        z_v[pl.ds(i, 16)] = jnp.zeros((16,), jnp.float32)

    pltpu.sync_copy(z_v, acc_sh.at[pl.ds(sid * (NPAD // NS), NPAD // NS)])
    pltpu.sync_copy(dst_hbm.at[wid], idx_v)
    plsc.subcore_barrier()

    @pl.loop(0, NCH)
    def _(j):
        pltpu.async_copy(ones_v, acc_sh.at[idx_v.at[j]], sem, add=True)

    @pl.loop(0, NCH)
    def _(j):
        pltpu.make_async_copy(ones_v, acc_sh.at[pl.ds(0, CNK)], sem).wait()
    plsc.subcore_barrier()

    s0 = sid * (NPAD // NS)
    pltpu.sync_copy(acc_sh.at[pl.ds(s0, NPAD // NS)],
                    out_hbm.at[cid, pl.ds(s0, NPAD // NS)])


@functools.partial(
    pl.kernel,
    out_type=jax.ShapeDtypeStruct((NC, NPAD, D), jnp.float32),
    mesh=_mesh,
    scratch_types=[
        pltpu.VMEM((NIB, SCH, CNK), jnp.int32),
        pltpu.VMEM((NIB, SCH, CNK), jnp.int32),
        pltpu.VMEM((NBUF, CNK, D), jnp.float32),
        pltpu.VMEM_SHARED((NPAD, D), jnp.float32),
        [pltpu.SemaphoreType.DMA] * NBUF,
        [pltpu.SemaphoreType.DMA] * NBUF,
    ],
)
def _sc_edge(g_hbm, src_hbm, dst_hbm, out_hbm, si_v, di_v, rows_v, acc_sh,
             gsems, ssems):
    cid = lax.axis_index("c")
    sid = lax.axis_index("s")
    wid = cid * NS + sid

    def start_gather(c, b):
        pltpu.async_copy(g_hbm.at[si_v.at[(c // SCH) % NIB, c % SCH]],
                         rows_v.at[b], gsems[b])

    def start_scatter(c, b):
        pltpu.async_copy(rows_v.at[b],
                         acc_sh.at[di_v.at[(c // SCH) % NIB, c % SCH]],
                         ssems[b], add=True)

    def wait(sem, b):
        pltpu.make_async_copy(g_hbm.at[pl.ds(0, CNK)], rows_v.at[b],
                              sem).wait()

    def load_idx(s, bi):
        pltpu.sync_copy(src_hbm.at[wid, pl.ds(s * SCH, SCH)], si_v.at[bi])
        pltpu.sync_copy(dst_hbm.at[wid, pl.ds(s * SCH, SCH)], di_v.at[bi])

    load_idx(0, 0)
    for b in range(1, NBUF):
        start_gather(b, b)

    @pl.loop(0, CNK)
    def _(r):
        for c in range(D // 16):
            rows_v[0, r, pl.ds(c * 16, 16)] = jnp.zeros((16,), jnp.float32)

    for k in range(NPAD // NS // CNK):
        pltpu.async_copy(rows_v.at[0],
                         acc_sh.at[pl.ds(sid * (NPAD // NS) + k * CNK, CNK)],
                         ssems[0])
    for k in range(NPAD // NS // CNK):
        pltpu.make_async_copy(g_hbm.at[pl.ds(0, CNK)], rows_v.at[0],
                              ssems[0]).wait()
    plsc.subcore_barrier()
    start_gather(0, 0)

    @pl.loop(0, NCH - NBUF, step=NBUF)
    def _(jj):
        @pl.when(jj % SCH == 0)
        def _():
            s1 = jj // SCH + 1

            @pl.when(s1 < NSUP)
            def _():
                load_idx(s1, s1 % NIB)

        for b in range(NBUF):
            wait(gsems[b], b)
            start_scatter(jj + b, b)
        for b in range(NBUF):
            wait(ssems[b], b)
            start_gather(jj + NBUF + b, b)

    for b in range(NBUF):
        wait(gsems[b], b)
        start_scatter(NCH - NBUF + b, b)
    for b in range(NBUF):
        wait(ssems[b], b)
    plsc.subcore_barrier()

    s0 = sid * (NPAD // NS)
    pltpu.sync_copy(acc_sh.at[pl.ds(s0, NPAD // NS)],
                    out_hbm.at[cid, pl.ds(s0, NPAD // NS)])


def _ln_matmul(h, scale, bias, w, dinv):
    mu = jnp.mean(h, axis=-1, keepdims=True)
    hc = h - mu
    var = jnp.mean(hc * hc, axis=-1, keepdims=True)
    ln = hc * lax.rsqrt(var + 1e-5) * scale + bias
    return jnp.dot(ln, w, preferred_element_type=jnp.float32) * dinv


def _dinv_of(deg_ref):
    d = deg_ref[0] + deg_ref[1] + 1.0
    return lax.rsqrt(d)[:, None]


def _pre0_body(x_ref, deg_ref, s_ref, b_ref, w_ref, o_ref):
    o_ref[...] = _ln_matmul(x_ref[...], s_ref[...], b_ref[...], w_ref[...],
                            _dinv_of(deg_ref))


def _layer_body(sp_ref, g_ref, deg_ref, bp_ref, s_ref, b_ref, w_ref, o_ref):
    dinv = _dinv_of(deg_ref)
    h = jnp.maximum((sp_ref[0] + sp_ref[1] + g_ref[...]) * dinv + bp_ref[...],
                    0.0)
    o_ref[...] = _ln_matmul(h, s_ref[...], b_ref[...], w_ref[...], dinv)


def _final_body(sp_ref, g_ref, deg_ref, bp_ref, batch_ref, lw_ref, lb_ref,
                o_ref, pooled, cnt):
    i = pl.program_id(0)

    @pl.when(i == 0)
    def _():
        pooled[...] = jnp.zeros_like(pooled)
        cnt[...] = jnp.zeros_like(cnt)

    dinv = _dinv_of(deg_ref)
    h = jnp.maximum((sp_ref[0] + sp_ref[1] + g_ref[...]) * dinv + bp_ref[...],
                    0.0)
    gids = lax.broadcasted_iota(jnp.int32, (G, 128), 0)
    for r in range(RB // 128):
        mask = (gids == batch_ref[0, r, :][None, :]).astype(jnp.float32)
        pooled[...] += jnp.dot(mask, h[r * 128:(r + 1) * 128, :],
                               preferred_element_type=jnp.float32)
        cnt[...] += jnp.sum(mask, axis=1, keepdims=True)

    @pl.when(i == GRID - 1)
    def _():
        p = pooled[...] / jnp.maximum(cnt[...], 1.0)
        o_ref[...] = (jnp.dot(p, lw_ref[...], preferred_element_type=jnp.float32)
                      + lb_ref[...])


def _row_spec():
    return pl.BlockSpec((RB, D), lambda i: (i, 0))


def _part_spec():
    return pl.BlockSpec((NC, RB, D), lambda i: (0, i, 0))


def _deg_spec():
    return pl.BlockSpec((NC, RB), lambda i: (0, i))


def _vec_spec():
    return pl.BlockSpec((1, D), lambda i: (0, 0))


def _mat_spec():
    return pl.BlockSpec((D, D), lambda i: (0, 0))


_pre0_call = pl.pallas_call(
    _pre0_body,
    grid=(GRID,),
    in_specs=[_row_spec(), _deg_spec(), _vec_spec(), _vec_spec(), _mat_spec()],
    out_specs=_row_spec(),
    out_shape=jax.ShapeDtypeStruct((NPAD, D), jnp.float32),
)

_layer_call = pl.pallas_call(
    _layer_body,
    grid=(GRID,),
    in_specs=[_part_spec(), _row_spec(), _deg_spec(), _vec_spec(), _vec_spec(),
              _vec_spec(), _mat_spec()],
    out_specs=_row_spec(),
    out_shape=jax.ShapeDtypeStruct((NPAD, D), jnp.float32),
)

_final_call = pl.pallas_call(
    _final_body,
    grid=(GRID,),
    in_specs=[_part_spec(), _row_spec(), _deg_spec(), _vec_spec(),
              pl.BlockSpec((1, RB // 128, 128), lambda i: (i, 0, 0)),
              _mat_spec(), _vec_spec()],
    out_specs=pl.BlockSpec((G, D), lambda i: (0, 0)),
    out_shape=jax.ShapeDtypeStruct((G, D), jnp.float32),
    scratch_shapes=[pltpu.VMEM((G, D), jnp.float32),
                    pltpu.VMEM((G, 1), jnp.float32)],
)


def kernel(x, edge_index, batch, ln_scale_0, ln_bias_0, W_0, b_0, ln_scale_1,
           ln_bias_1, W_1, b_1, ln_scale_2, ln_bias_2, W_2, b_2, lin_W, lin_b):
    pad_i = jnp.arange(E_PAD - E, dtype=jnp.int32)
    src = jnp.concatenate([edge_index[0], (pad_i * 131) % N])
    dst = jnp.concatenate([edge_index[1], N + (pad_i % (NPAD - N - 8))])
    src3 = src.reshape(NCH, NW, CNK).transpose(1, 0, 2)
    dst3 = dst.reshape(NCH, NW, CNK).transpose(1, 0, 2)
    x_pad = jnp.pad(x, ((0, NPAD - N), (0, 0)))
    batch3d = jnp.pad(batch, (0, NPAD - N),
                      constant_values=G + 7).reshape(GRID, RB // 128, 128)

    deg = _sc_deg(dst3)

    params = [(ln_scale_0.reshape(1, D), ln_bias_0.reshape(1, D), W_0,
               b_0.reshape(1, D)),
              (ln_scale_1.reshape(1, D), ln_bias_1.reshape(1, D), W_1,
               b_1.reshape(1, D)),
              (ln_scale_2.reshape(1, D), ln_bias_2.reshape(1, D), W_2,
               b_2.reshape(1, D))]

    g = _pre0_call(x_pad, deg, params[0][0], params[0][1], params[0][2])
    for l in (1, 2):
        s_part = _sc_edge(g, src3, dst3)
        g = _layer_call(s_part, g, deg, params[l - 1][3], params[l][0],
                        params[l][1], params[l][2])
    s_part = _sc_edge(g, src3, dst3)
    return _final_call(s_part, g, deg, params[2][3], batch3d, lin_W,
                       lin_b.reshape(1, D))
